# baseline (speedup 1.0000x reference)
import functools

import jax
import jax.numpy as jnp
from jax import lax
from jax.experimental import pallas as pl
from jax.experimental.pallas import tpu as pltpu
from jax.experimental.pallas import tpu_sc as plsc

N = 10000
NPAD = 10112
E = 320000
NS = 16
CH = 128
EPAD = 327680
PER_SUB = EPAD // NS
ITERS = PER_SUB // CH
RPS = NPAD // NS
CW = 8
HID = 128


NSLOT = 2
NB = 8
ROWS_SUB = ITERS


def _pad_edges(ei):
    pad = EPAD - E
    src = jnp.concatenate([ei[0], jnp.zeros((pad,), jnp.int32)])
    dst = jnp.concatenate([ei[1], jnp.full((pad,), N, jnp.int32)])
    return src.reshape(EPAD // CH, CH), dst.reshape(EPAD // CH, CH)


def _mesh():
    return plsc.VectorSubcoreMesh(core_axis_name="c", subcore_axis_name="s")


def _segsum_pair(tab0, s0, d0, tab1, s1, d1, with_counts=False):
    zz = jnp.zeros((NPAD, HID), jnp.float32)
    nblocks = ITERS // NB
    nsuper = nblocks // 2

    nidx = 1 if with_counts else 2
    nb = 4 if with_counts else NB
    scratch = (
        [pltpu.VMEM((nb, CH), jnp.int32)] * nidx +
        [pltpu.VMEM((nb, CH), jnp.int32)] * nidx +
        [
            pltpu.VMEM((NSLOT, CH, HID), jnp.float32),
            pltpu.VMEM_SHARED((NPAD, HID), jnp.float32),
        ] + [pltpu.SemaphoreType.DMA] * (2 * NSLOT + 2))
    if with_counts:
        scratch += [
            pltpu.VMEM((CH, CW), jnp.float32),
            pltpu.VMEM_SHARED((NPAD, CW), jnp.float32),
            pltpu.SemaphoreType.DMA,
        ]

    def body(t0, e_s0, e_d0, t1, e_s1, e_d1, zz0, out0, out1, *rest):
        if with_counts:
            cnt0, cnt1 = rest[0:2]
            rest = rest[2:]
        idx_s = rest[0:nidx]
        idx_d = rest[nidx:2 * nidx]
        rows = rest[2 * nidx]
        acc = rest[2 * nidx + 1]
        o = 2 * nidx + 2
        gsem = rest[o:o + NSLOT]
        ssem = rest[o + NSLOT:o + 2 * NSLOT]
        isem = rest[o + 2 * NSLOT:o + 2 * NSLOT + 2]
        if with_counts:
            ones_v, acc_cnt, csem = rest[o + 2 * NSLOT + 2:o + 2 * NSLOT + 5]
        cid = lax.axis_index("c")
        sid = lax.axis_index("s")
        rslice = pl.ds(sid * RPS, RPS)

        pltpu.sync_copy(zz0.at[rslice], acc.at[rslice])
        if with_counts:
            def fill(val):
                vec = jnp.full((CW,), val, jnp.float32)

                def store(r, c):
                    ones_v[r, :] = vec
                    return c
                lax.fori_loop(0, CH, store, 0)

            fill(0.0)
            r0 = sid * RPS
            for o in (0, CH, 2 * CH, 3 * CH):
                pltpu.sync_copy(ones_v, acc_cnt.at[pl.ds(r0 + o, CH)])
            pltpu.sync_copy(ones_v.at[pl.ds(0, RPS - 4 * CH)],
                            acc_cnt.at[pl.ds(r0 + 4 * CH, RPS - 4 * CH)])
            fill(1.0)
        plsc.subcore_barrier()

        def run_dir(es, ed, tab):
            base = sid * ROWS_SUB

            def load_block(b, p):
                pltpu.async_copy(es.at[pl.ds(base + b * NB, NB)], idx_s[p],
                                 isem[p])
                pltpu.async_copy(ed.at[pl.ds(base + b * NB, NB)], idx_d[p],
                                 isem[p])

            def wait_block(b, p):
                pltpu.make_async_copy(es.at[pl.ds(base + b * NB, NB)],
                                      idx_s[p], isem[p]).wait()
                pltpu.make_async_copy(ed.at[pl.ds(base + b * NB, NB)],
                                      idx_d[p], isem[p]).wait()

            def process(p):
                def gather(k, j):
                    return pltpu.make_async_copy(tab.at[idx_s[p].at[k]],
                                                 rows.at[j], gsem[j])

                def scatter(k, j):
                    return pltpu.make_async_copy(rows.at[j],
                                                 acc.at[idx_d[p].at[k]],
                                                 ssem[j])

                for j in range(NSLOT):
                    gather(j, j).start()
                for k in range(nb):
                    j = k % NSLOT
                    gather(k, j).wait()
                    pltpu.async_copy(rows.at[j], acc.at[idx_d[p].at[k]],
                                     ssem[j], add=True)
                    if with_counts:
                        pltpu.async_copy(ones_v, acc_cnt.at[idx_d[p].at[k]],
                                         csem, add=True)
                    if k + NSLOT < nb:
                        scatter(k, j).wait()
                        gather(k + NSLOT, j).start()
                for k in range(nb - NSLOT, nb):
                    scatter(k, k % NSLOT).wait()
                if with_counts:
                    for k in range(nb):
                        pltpu.make_async_copy(ones_v,
                                              acc_cnt.at[idx_d[p].at[k]],
                                              csem).wait()

            if with_counts:
                nblk4 = ITERS // 4
                base4 = sid * nblk4

                def sup1(b, carry):
                    pltpu.sync_copy(es.at[base4 + b], idx_s[0])
                    pltpu.sync_copy(ed.at[base4 + b], idx_d[0])
                    process(0)
                    return carry

                lax.fori_loop(0, nblk4, sup1, 0)
            else:
                load_block(0, 0)

                def sup(t, carry):
                    b0 = 2 * t
                    wait_block(b0, 0)
                    load_block(b0 + 1, 1)
                    process(0)
                    wait_block(b0 + 1, 1)

                    @pl.when(t + 1 < nsuper)
                    def _():
                        load_block(b0 + 2, 0)
                    process(1)
                    return carry

                lax.fori_loop(0, nsuper, sup, 0)

        @pl.when(cid == 0)
        def _():
            run_dir(e_s0, e_d0, t0)

        @pl.when(cid == 1)
        def _():
            run_dir(e_s1, e_d1, t1)

        plsc.subcore_barrier()

        @pl.when(cid == 0)
        def _():
            pltpu.sync_copy(acc.at[rslice], out0.at[rslice])
            if with_counts:
                pltpu.sync_copy(acc_cnt.at[rslice], cnt0.at[rslice])

        @pl.when(cid == 1)
        def _():
            pltpu.sync_copy(acc.at[rslice], out1.at[rslice])
            if with_counts:
                pltpu.sync_copy(acc_cnt.at[rslice], cnt1.at[rslice])

    out_type = [
        jax.ShapeDtypeStruct((NPAD, HID), jnp.float32),
        jax.ShapeDtypeStruct((NPAD, HID), jnp.float32),
    ]
    if with_counts:
        out_type += [jax.ShapeDtypeStruct((NPAD, CW), jnp.float32)] * 2

    run = pl.kernel(
        body,
        out_type=out_type,
        mesh=_mesh(),
        scratch_types=scratch,
    )
    return run(tab0, s0, d0, tab1, s1, d1, zz)


def _counts_pair(d0, d1):
    ones = jnp.ones((CH, HID), jnp.float32)
    zz = jnp.zeros((NPAD, HID), jnp.float32)
    nblocks = ITERS // NB
    nsuper = nblocks // 2

    scratch = [
        pltpu.VMEM((NB, CH), jnp.int32),
        pltpu.VMEM((NB, CH), jnp.int32),
        pltpu.VMEM((CH, HID), jnp.float32),
        pltpu.VMEM_SHARED((NPAD, HID), jnp.float32),
    ] + [pltpu.SemaphoreType.DMA] * 3

    def body(e_d0, e_d1, ones_h, zz0, out0, out1, *rest):
        idx_d = rest[0:2]
        ones_v, acc = rest[2], rest[3]
        ssem, isem0, isem1 = rest[4:7]
        isem = (isem0, isem1)
        cid = lax.axis_index("c")
        sid = lax.axis_index("s")
        rslice = pl.ds(sid * RPS, RPS)

        pltpu.sync_copy(zz0.at[rslice], acc.at[rslice])
        pltpu.sync_copy(ones_h, ones_v)
        plsc.subcore_barrier()

        def run_dir(ed):
            base = sid * ROWS_SUB

            def load_block(b, p):
                pltpu.async_copy(ed.at[pl.ds(base + b * NB, NB)], idx_d[p],
                                 isem[p])

            def wait_block(b, p):
                pltpu.make_async_copy(ed.at[pl.ds(base + b * NB, NB)],
                                      idx_d[p], isem[p]).wait()

            def process(p):
                for k in range(NB):
                    pltpu.async_copy(ones_v, acc.at[idx_d[p].at[k]],
                                     ssem, add=True)
                for k in range(NB):
                    pltpu.make_async_copy(ones_v, acc.at[idx_d[p].at[k]],
                                          ssem).wait()

            load_block(0, 0)

            def sup(t, carry):
                b0 = 2 * t
                wait_block(b0, 0)
                load_block(b0 + 1, 1)
                process(0)
                wait_block(b0 + 1, 1)

                @pl.when(t + 1 < nsuper)
                def _():
                    load_block(b0 + 2, 0)
                process(1)
                return carry

            lax.fori_loop(0, nsuper, sup, 0)

        @pl.when(cid == 0)
        def _():
            run_dir(e_d0)

        @pl.when(cid == 1)
        def _():
            run_dir(e_d1)

        plsc.subcore_barrier()

        @pl.when(cid == 0)
        def _():
            pltpu.sync_copy(acc.at[rslice], out0.at[rslice])

        @pl.when(cid == 1)
        def _():
            pltpu.sync_copy(acc.at[rslice], out1.at[rslice])

    run = pl.kernel(
        body,
        out_type=[
            jax.ShapeDtypeStruct((NPAD, HID), jnp.float32),
            jax.ShapeDtypeStruct((NPAD, HID), jnp.float32),
        ],
        mesh=_mesh(),
        scratch_types=scratch,
    )
    return run(d0, d1, ones, zz)


BM = 1000


def _row(d):
    return pl.BlockSpec((BM, d), lambda i: (i, 0))


def _full(a):
    return pl.BlockSpec(a.shape, lambda i: (0,) * a.ndim)


def _dense_mid(sum_i, cnt_i, h_i, wl_ui, wr_ui, b_ui,
               sum_u, cnt_u, h_u, wl_iu, wr_iu, b_iu):
    Du = sum_i.shape[1]
    Di = sum_u.shape[1]

    def body(si, ci, hi, wlui, wrui, bui, su, cu, hu, wliu, wriu, biu, oi, ou):
        mi = si[...] * (1.0 / jnp.maximum(ci[...][:, :1], 1.0))
        vi = (jnp.dot(mi, wlui[...], preferred_element_type=jnp.float32)
              + jnp.dot(hi[...], wrui[...], preferred_element_type=jnp.float32)
              + bui[...])
        oi[...] = jnp.maximum(vi, 0.0)
        mu = su[...] * (1.0 / jnp.maximum(cu[...][:, :1], 1.0))
        vu = (jnp.dot(mu, wliu[...], preferred_element_type=jnp.float32)
              + jnp.dot(hu[...], wriu[...], preferred_element_type=jnp.float32)
              + biu[...])
        ou[...] = jnp.maximum(vu, 0.0)

    args = (sum_i, cnt_i, h_i, wl_ui, wr_ui, b_ui,
            sum_u, cnt_u, h_u, wl_iu, wr_iu, b_iu)
    specs = [_row(Du), _row(CW), _row(Di), _full(wl_ui), _full(wr_ui), _full(b_ui),
             _row(Di), _row(CW), _row(Du), _full(wl_iu), _full(wr_iu), _full(b_iu)]
    return pl.pallas_call(
        body,
        grid=(N // BM,),
        in_specs=specs,
        out_specs=[_row(HID), _row(HID)],
        out_shape=[jax.ShapeDtypeStruct((N, HID), jnp.float32)] * 2,
    )(*args)


def _dense_final(sum_i, cnt_i, h_i, wl_ui, wr_ui, b_ui,
                 sum_u, cnt_u, h_u, wl_iu, wr_iu, b_iu,
                 wh_u, bh_u, wh_i, bh_i):
    Du = sum_i.shape[1]
    Di = sum_u.shape[1]

    def body(si, ci, hi, wlui, wrui, bui, su, cu, hu, wliu, wriu, biu,
             whu, bhu, whi, bhi, zu, zi):
        mi = si[...] * (1.0 / jnp.maximum(ci[...][:, :1], 1.0))
        vi = (jnp.dot(mi, wlui[...], preferred_element_type=jnp.float32)
              + jnp.dot(hi[...], wrui[...], preferred_element_type=jnp.float32)
              + bui[...])
        hi_new = jnp.maximum(vi, 0.0)
        zi[...] = jnp.dot(hi_new, whi[...], preferred_element_type=jnp.float32) + bhi[...]
        mu = su[...] * (1.0 / jnp.maximum(cu[...][:, :1], 1.0))
        vu = (jnp.dot(mu, wliu[...], preferred_element_type=jnp.float32)
              + jnp.dot(hu[...], wriu[...], preferred_element_type=jnp.float32)
              + biu[...])
        hu_new = jnp.maximum(vu, 0.0)
        zu[...] = jnp.dot(hu_new, whu[...], preferred_element_type=jnp.float32) + bhu[...]

    args = (sum_i, cnt_i, h_i, wl_ui, wr_ui, b_ui,
            sum_u, cnt_u, h_u, wl_iu, wr_iu, b_iu,
            wh_u, bh_u, wh_i, bh_i)
    specs = [_row(Du), _row(CW), _row(Di), _full(wl_ui), _full(wr_ui), _full(b_ui),
             _row(Di), _row(CW), _row(Du), _full(wl_iu), _full(wr_iu), _full(b_iu),
             _full(wh_u), _full(bh_u), _full(wh_i), _full(bh_i)]
    return pl.pallas_call(
        body,
        grid=(N // BM,),
        in_specs=specs,
        out_specs=[_row(HID), _row(HID)],
        out_shape=[jax.ShapeDtypeStruct((N, HID), jnp.float32)] * 2,
    )(*args)


def kernel(x_user, edge_index_ui, edge_index_iu, params):
    s_ui, d_ui = _pad_edges(edge_index_ui)
    s_iu, d_iu = _pad_edges(edge_index_iu)

    cnt_i, cnt_u = _counts_pair(d_ui, d_iu)
    cnt_i = cnt_i[:N, :CW]
    cnt_u = cnt_u[:N, :CW]

    h_u = x_user
    h_i = params["emb_item"]
    layers = params["layers"]

    for li in range(len(layers)):
        p = layers[li]
        Di = h_i.shape[1]
        tab_i = h_i if Di == HID else jnp.pad(h_i, ((0, 0), (0, HID - Di)))
        sum_i, sum_u = _segsum_pair(h_u, s_ui, d_ui, tab_i, s_iu, d_iu)
        sum_i = sum_i[:N]
        sum_u = sum_u[:N, :Di]
        a = (sum_i, cnt_i, h_i, p["ui"]["W_l"].T, p["ui"]["W_r"].T,
             p["ui"]["b_l"][None, :],
             sum_u, cnt_u, h_u, p["iu"]["W_l"].T, p["iu"]["W_r"].T,
             p["iu"]["b_l"][None, :])
        if li + 1 < len(layers):
            h_i, h_u = _dense_mid(*a)
        else:
            z_u, z_i = _dense_final(
                *a,
                params["lin_user"]["W"].T, params["lin_user"]["b"][None, :],
                params["lin_item"]["W"].T, params["lin_item"]["b"][None, :])
    return z_u, z_i


# speedup vs baseline: 1.0004x; 1.0004x over previous
import jax
import jax.numpy as jnp
from jax import lax
from jax.experimental import pallas as pl
from jax.experimental.pallas import tpu as pltpu
from jax.experimental.pallas import tpu_sc as plsc

N = 10000
NPAD = 10112
E = 320000
NS = 16
CH = 128
EPAD = 327680
PER_SUB = EPAD // NS
ITERS = PER_SUB // CH
RPS = NPAD // NS
CW = 8
HID = 128


NSLOT = 2
NB = 8
ROWS_SUB = ITERS


def _pad_edges(ei):
    pad = EPAD - E
    src = jnp.concatenate([ei[0], jnp.zeros((pad,), jnp.int32)])
    dst = jnp.concatenate([ei[1], jnp.full((pad,), N, jnp.int32)])
    return src.reshape(EPAD // CH, CH), dst.reshape(EPAD // CH, CH)


def _mesh():
    return plsc.VectorSubcoreMesh(core_axis_name="c", subcore_axis_name="s")


def _segsum_pair(tab0, s0, d0, tab1, s1, d1):
    zz = jnp.zeros((NPAD, HID), jnp.float32)
    nblocks = ITERS // NB
    nsuper = nblocks // 2

    scratch = [
        pltpu.VMEM((NB, CH), jnp.int32),
        pltpu.VMEM((NB, CH), jnp.int32),
        pltpu.VMEM((NB, CH), jnp.int32),
        pltpu.VMEM((NB, CH), jnp.int32),
        pltpu.VMEM((NSLOT, CH, HID), jnp.float32),
        pltpu.VMEM_SHARED((NPAD, HID), jnp.float32),
    ] + [pltpu.SemaphoreType.DMA] * (2 * NSLOT + 2)

    def body(t0, e_s0, e_d0, t1, e_s1, e_d1, zz0, out0, out1, *rest):
        idx_s = rest[0:2]
        idx_d = rest[2:4]
        rows = rest[4]
        acc = rest[5]
        gsem = rest[6:6 + NSLOT]
        ssem = rest[6 + NSLOT:6 + 2 * NSLOT]
        isem = rest[6 + 2 * NSLOT:6 + 2 * NSLOT + 2]
        cid = lax.axis_index("c")
        sid = lax.axis_index("s")
        rslice = pl.ds(sid * RPS, RPS)

        pltpu.sync_copy(zz0.at[rslice], acc.at[rslice])
        plsc.subcore_barrier()

        def run_dir(es, ed, tab):
            base = sid * ROWS_SUB

            def load_block(b, p):
                pltpu.async_copy(es.at[pl.ds(base + b * NB, NB)], idx_s[p],
                                 isem[p])
                pltpu.async_copy(ed.at[pl.ds(base + b * NB, NB)], idx_d[p],
                                 isem[p])

            def wait_block(b, p):
                pltpu.make_async_copy(es.at[pl.ds(base + b * NB, NB)],
                                      idx_s[p], isem[p]).wait()
                pltpu.make_async_copy(ed.at[pl.ds(base + b * NB, NB)],
                                      idx_d[p], isem[p]).wait()

            def process(p):
                def gather(k, j):
                    return pltpu.make_async_copy(tab.at[idx_s[p].at[k]],
                                                 rows.at[j], gsem[j])

                def scatter(k, j):
                    return pltpu.make_async_copy(rows.at[j],
                                                 acc.at[idx_d[p].at[k]],
                                                 ssem[j])

                for j in range(NSLOT):
                    gather(j, j).start()
                for k in range(NB):
                    j = k % NSLOT
                    gather(k, j).wait()
                    pltpu.async_copy(rows.at[j], acc.at[idx_d[p].at[k]],
                                     ssem[j], add=True)
                    if k + NSLOT < NB:
                        scatter(k, j).wait()
                        gather(k + NSLOT, j).start()
                for k in range(NB - NSLOT, NB):
                    scatter(k, k % NSLOT).wait()

            load_block(0, 0)

            def sup(t, carry):
                b0 = 2 * t
                wait_block(b0, 0)
                load_block(b0 + 1, 1)
                process(0)
                wait_block(b0 + 1, 1)

                @pl.when(t + 1 < nsuper)
                def _():
                    load_block(b0 + 2, 0)
                process(1)
                return carry

            lax.fori_loop(0, nsuper, sup, 0)

        @pl.when(cid == 0)
        def _():
            run_dir(e_s0, e_d0, t0)

        @pl.when(cid == 1)
        def _():
            run_dir(e_s1, e_d1, t1)

        plsc.subcore_barrier()

        @pl.when(cid == 0)
        def _():
            pltpu.sync_copy(acc.at[rslice], out0.at[rslice])

        @pl.when(cid == 1)
        def _():
            pltpu.sync_copy(acc.at[rslice], out1.at[rslice])

    run = pl.kernel(
        body,
        out_type=[
            jax.ShapeDtypeStruct((NPAD, HID), jnp.float32),
            jax.ShapeDtypeStruct((NPAD, HID), jnp.float32),
        ],
        mesh=_mesh(),
        scratch_types=scratch,
    )
    return run(tab0, s0, d0, tab1, s1, d1, zz)


def _counts_pair(d0, d1):
    ones = jnp.ones((CH, HID), jnp.float32)
    zz = jnp.zeros((NPAD, HID), jnp.float32)
    nblocks = ITERS // NB
    nsuper = nblocks // 2

    scratch = [
        pltpu.VMEM((NB, CH), jnp.int32),
        pltpu.VMEM((NB, CH), jnp.int32),
        pltpu.VMEM((CH, HID), jnp.float32),
        pltpu.VMEM_SHARED((NPAD, HID), jnp.float32),
    ] + [pltpu.SemaphoreType.DMA] * 3

    def body(e_d0, e_d1, ones_h, zz0, out0, out1, *rest):
        idx_d = rest[0:2]
        ones_v, acc = rest[2], rest[3]
        ssem, isem0, isem1 = rest[4:7]
        isem = (isem0, isem1)
        cid = lax.axis_index("c")
        sid = lax.axis_index("s")
        rslice = pl.ds(sid * RPS, RPS)

        pltpu.sync_copy(zz0.at[rslice], acc.at[rslice])
        pltpu.sync_copy(ones_h, ones_v)
        plsc.subcore_barrier()

        def run_dir(ed):
            base = sid * ROWS_SUB

            def load_block(b, p):
                pltpu.async_copy(ed.at[pl.ds(base + b * NB, NB)], idx_d[p],
                                 isem[p])

            def wait_block(b, p):
                pltpu.make_async_copy(ed.at[pl.ds(base + b * NB, NB)],
                                      idx_d[p], isem[p]).wait()

            def process(p):
                for k in range(NB):
                    pltpu.async_copy(ones_v, acc.at[idx_d[p].at[k]],
                                     ssem, add=True)
                for k in range(NB):
                    pltpu.make_async_copy(ones_v, acc.at[idx_d[p].at[k]],
                                          ssem).wait()

            load_block(0, 0)

            def sup(t, carry):
                b0 = 2 * t
                wait_block(b0, 0)
                load_block(b0 + 1, 1)
                process(0)
                wait_block(b0 + 1, 1)

                @pl.when(t + 1 < nsuper)
                def _():
                    load_block(b0 + 2, 0)
                process(1)
                return carry

            lax.fori_loop(0, nsuper, sup, 0)

        @pl.when(cid == 0)
        def _():
            run_dir(e_d0)

        @pl.when(cid == 1)
        def _():
            run_dir(e_d1)

        plsc.subcore_barrier()

        @pl.when(cid == 0)
        def _():
            pltpu.sync_copy(acc.at[rslice], out0.at[rslice])

        @pl.when(cid == 1)
        def _():
            pltpu.sync_copy(acc.at[rslice], out1.at[rslice])

    run = pl.kernel(
        body,
        out_type=[
            jax.ShapeDtypeStruct((NPAD, HID), jnp.float32),
            jax.ShapeDtypeStruct((NPAD, HID), jnp.float32),
        ],
        mesh=_mesh(),
        scratch_types=scratch,
    )
    return run(d0, d1, ones, zz)


BM = 1000


def _row(d):
    return pl.BlockSpec((BM, d), lambda i: (i, 0))


def _full(a):
    return pl.BlockSpec(a.shape, lambda i: (0,) * a.ndim)


def _dense_mid(sum_i, cnt_i, h_i, wl_ui, wr_ui, b_ui,
               sum_u, cnt_u, h_u, wl_iu, wr_iu, b_iu):
    Du = sum_i.shape[1]
    Di = sum_u.shape[1]

    def body(si, ci, hi, wlui, wrui, bui, su, cu, hu, wliu, wriu, biu, oi, ou):
        mi = si[...] * (1.0 / jnp.maximum(ci[...][:, :1], 1.0))
        vi = (jnp.dot(mi, wlui[...], preferred_element_type=jnp.float32)
              + jnp.dot(hi[...], wrui[...], preferred_element_type=jnp.float32)
              + bui[...])
        oi[...] = jnp.maximum(vi, 0.0)
        mu = su[...] * (1.0 / jnp.maximum(cu[...][:, :1], 1.0))
        vu = (jnp.dot(mu, wliu[...], preferred_element_type=jnp.float32)
              + jnp.dot(hu[...], wriu[...], preferred_element_type=jnp.float32)
              + biu[...])
        ou[...] = jnp.maximum(vu, 0.0)

    args = (sum_i, cnt_i, h_i, wl_ui, wr_ui, b_ui,
            sum_u, cnt_u, h_u, wl_iu, wr_iu, b_iu)
    specs = [_row(Du), _row(CW), _row(Di), _full(wl_ui), _full(wr_ui), _full(b_ui),
             _row(Di), _row(CW), _row(Du), _full(wl_iu), _full(wr_iu), _full(b_iu)]
    return pl.pallas_call(
        body,
        grid=(N // BM,),
        in_specs=specs,
        out_specs=[_row(HID), _row(HID)],
        out_shape=[jax.ShapeDtypeStruct((N, HID), jnp.float32)] * 2,
    )(*args)


def _dense_final(sum_i, cnt_i, h_i, wl_ui, wr_ui, b_ui,
                 sum_u, cnt_u, h_u, wl_iu, wr_iu, b_iu,
                 wh_u, bh_u, wh_i, bh_i):
    Du = sum_i.shape[1]
    Di = sum_u.shape[1]

    def body(si, ci, hi, wlui, wrui, bui, su, cu, hu, wliu, wriu, biu,
             whu, bhu, whi, bhi, zu, zi):
        mi = si[...] * (1.0 / jnp.maximum(ci[...][:, :1], 1.0))
        vi = (jnp.dot(mi, wlui[...], preferred_element_type=jnp.float32)
              + jnp.dot(hi[...], wrui[...], preferred_element_type=jnp.float32)
              + bui[...])
        hi_new = jnp.maximum(vi, 0.0)
        zi[...] = jnp.dot(hi_new, whi[...], preferred_element_type=jnp.float32) + bhi[...]
        mu = su[...] * (1.0 / jnp.maximum(cu[...][:, :1], 1.0))
        vu = (jnp.dot(mu, wliu[...], preferred_element_type=jnp.float32)
              + jnp.dot(hu[...], wriu[...], preferred_element_type=jnp.float32)
              + biu[...])
        hu_new = jnp.maximum(vu, 0.0)
        zu[...] = jnp.dot(hu_new, whu[...], preferred_element_type=jnp.float32) + bhu[...]

    args = (sum_i, cnt_i, h_i, wl_ui, wr_ui, b_ui,
            sum_u, cnt_u, h_u, wl_iu, wr_iu, b_iu,
            wh_u, bh_u, wh_i, bh_i)
    specs = [_row(Du), _row(CW), _row(Di), _full(wl_ui), _full(wr_ui), _full(b_ui),
             _row(Di), _row(CW), _row(Du), _full(wl_iu), _full(wr_iu), _full(b_iu),
             _full(wh_u), _full(bh_u), _full(wh_i), _full(bh_i)]
    return pl.pallas_call(
        body,
        grid=(N // BM,),
        in_specs=specs,
        out_specs=[_row(HID), _row(HID)],
        out_shape=[jax.ShapeDtypeStruct((N, HID), jnp.float32)] * 2,
    )(*args)


def kernel(x_user, edge_index_ui, edge_index_iu, params):
    s_ui, d_ui = _pad_edges(edge_index_ui)
    s_iu, d_iu = _pad_edges(edge_index_iu)

    cnt_i, cnt_u = _counts_pair(d_ui, d_iu)
    cnt_i = cnt_i[:N, :CW]
    cnt_u = cnt_u[:N, :CW]

    h_u = x_user
    h_i = params["emb_item"]
    layers = params["layers"]

    for li in range(len(layers)):
        p = layers[li]
        Di = h_i.shape[1]
        tab_i = h_i if Di == HID else jnp.pad(h_i, ((0, 0), (0, HID - Di)))
        sum_i, sum_u = _segsum_pair(h_u, s_ui, d_ui, tab_i, s_iu, d_iu)
        sum_i = sum_i[:N]
        sum_u = sum_u[:N, :Di]
        a = (sum_i, cnt_i, h_i, p["ui"]["W_l"].T, p["ui"]["W_r"].T,
             p["ui"]["b_l"][None, :],
             sum_u, cnt_u, h_u, p["iu"]["W_l"].T, p["iu"]["W_r"].T,
             p["iu"]["b_l"][None, :])
        if li + 1 < len(layers):
            h_i, h_u = _dense_mid(*a)
        else:
            z_u, z_i = _dense_final(
                *a,
                params["lin_user"]["W"].T, params["lin_user"]["b"][None, :],
                params["lin_item"]["W"].T, params["lin_item"]["b"][None, :])
    return z_u, z_i


# speedup vs baseline: 1.0172x; 1.0168x over previous
import jax
import jax.numpy as jnp
from jax import lax
from jax.experimental import pallas as pl
from jax.experimental.pallas import tpu as pltpu
from jax.experimental.pallas import tpu_sc as plsc

N = 10000
NPAD = 10112
E = 320000
NS = 16
CH = 128
EPAD = 327680
PER_SUB = EPAD // NS
ITERS = PER_SUB // CH
RPS = NPAD // NS
CW = 8
HID = 128


NSLOT = 2
NB = 16
ROWS_SUB = ITERS


def _pad_edges(ei):
    pad = EPAD - E
    src = jnp.concatenate([ei[0], jnp.zeros((pad,), jnp.int32)])
    dst = jnp.concatenate([ei[1], jnp.full((pad,), N, jnp.int32)])
    return src.reshape(EPAD // CH, CH), dst.reshape(EPAD // CH, CH)


def _mesh():
    return plsc.VectorSubcoreMesh(core_axis_name="c", subcore_axis_name="s")


def _segsum_pair(tab0, s0, d0, tab1, s1, d1):
    zz = jnp.zeros((NPAD, HID), jnp.float32)
    nblocks = ITERS // NB
    nsuper = nblocks // 2

    scratch = [
        pltpu.VMEM((NB, CH), jnp.int32),
        pltpu.VMEM((NB, CH), jnp.int32),
        pltpu.VMEM((NB, CH), jnp.int32),
        pltpu.VMEM((NB, CH), jnp.int32),
        pltpu.VMEM((NSLOT, CH, HID), jnp.float32),
        pltpu.VMEM_SHARED((NPAD, HID), jnp.float32),
    ] + [pltpu.SemaphoreType.DMA] * (2 * NSLOT + 2)

    def body(t0, e_s0, e_d0, t1, e_s1, e_d1, zz0, out0, out1, *rest):
        idx_s = rest[0:2]
        idx_d = rest[2:4]
        rows = rest[4]
        acc = rest[5]
        gsem = rest[6:6 + NSLOT]
        ssem = rest[6 + NSLOT:6 + 2 * NSLOT]
        isem = rest[6 + 2 * NSLOT:6 + 2 * NSLOT + 2]
        cid = lax.axis_index("c")
        sid = lax.axis_index("s")
        rslice = pl.ds(sid * RPS, RPS)

        pltpu.sync_copy(zz0.at[rslice], acc.at[rslice])
        plsc.subcore_barrier()

        def run_dir(es, ed, tab):
            base = sid * ROWS_SUB

            def load_block(b, p):
                pltpu.async_copy(es.at[pl.ds(base + b * NB, NB)], idx_s[p],
                                 isem[p])
                pltpu.async_copy(ed.at[pl.ds(base + b * NB, NB)], idx_d[p],
                                 isem[p])

            def wait_block(b, p):
                pltpu.make_async_copy(es.at[pl.ds(base + b * NB, NB)],
                                      idx_s[p], isem[p]).wait()
                pltpu.make_async_copy(ed.at[pl.ds(base + b * NB, NB)],
                                      idx_d[p], isem[p]).wait()

            def process(p):
                def gather(k, j):
                    return pltpu.make_async_copy(tab.at[idx_s[p].at[k]],
                                                 rows.at[j], gsem[j])

                def scatter(k, j):
                    return pltpu.make_async_copy(rows.at[j],
                                                 acc.at[idx_d[p].at[k]],
                                                 ssem[j])

                for j in range(NSLOT):
                    gather(j, j).start()
                for k in range(NB):
                    j = k % NSLOT
                    gather(k, j).wait()
                    pltpu.async_copy(rows.at[j], acc.at[idx_d[p].at[k]],
                                     ssem[j], add=True)
                    if k + NSLOT < NB:
                        scatter(k, j).wait()
                        gather(k + NSLOT, j).start()
                for k in range(NB - NSLOT, NB):
                    scatter(k, k % NSLOT).wait()

            load_block(0, 0)

            def sup(t, carry):
                b0 = 2 * t
                wait_block(b0, 0)
                load_block(b0 + 1, 1)
                process(0)
                wait_block(b0 + 1, 1)

                @pl.when(t + 1 < nsuper)
                def _():
                    load_block(b0 + 2, 0)
                process(1)
                return carry

            lax.fori_loop(0, nsuper, sup, 0)

        @pl.when(cid == 0)
        def _():
            run_dir(e_s0, e_d0, t0)

        @pl.when(cid == 1)
        def _():
            run_dir(e_s1, e_d1, t1)

        plsc.subcore_barrier()

        @pl.when(cid == 0)
        def _():
            pltpu.sync_copy(acc.at[rslice], out0.at[rslice])

        @pl.when(cid == 1)
        def _():
            pltpu.sync_copy(acc.at[rslice], out1.at[rslice])

    run = pl.kernel(
        body,
        out_type=[
            jax.ShapeDtypeStruct((NPAD, HID), jnp.float32),
            jax.ShapeDtypeStruct((NPAD, HID), jnp.float32),
        ],
        mesh=_mesh(),
        scratch_types=scratch,
    )
    return run(tab0, s0, d0, tab1, s1, d1, zz)


def _counts_pair(d0, d1):
    ones = jnp.ones((CH, HID), jnp.float32)
    zz = jnp.zeros((NPAD, HID), jnp.float32)
    nblocks = ITERS // NB
    nsuper = nblocks // 2

    scratch = [
        pltpu.VMEM((NB, CH), jnp.int32),
        pltpu.VMEM((NB, CH), jnp.int32),
        pltpu.VMEM((CH, HID), jnp.float32),
        pltpu.VMEM_SHARED((NPAD, HID), jnp.float32),
    ] + [pltpu.SemaphoreType.DMA] * 3

    def body(e_d0, e_d1, ones_h, zz0, out0, out1, *rest):
        idx_d = rest[0:2]
        ones_v, acc = rest[2], rest[3]
        ssem, isem0, isem1 = rest[4:7]
        isem = (isem0, isem1)
        cid = lax.axis_index("c")
        sid = lax.axis_index("s")
        rslice = pl.ds(sid * RPS, RPS)

        pltpu.sync_copy(zz0.at[rslice], acc.at[rslice])
        pltpu.sync_copy(ones_h, ones_v)
        plsc.subcore_barrier()

        def run_dir(ed):
            base = sid * ROWS_SUB

            def load_block(b, p):
                pltpu.async_copy(ed.at[pl.ds(base + b * NB, NB)], idx_d[p],
                                 isem[p])

            def wait_block(b, p):
                pltpu.make_async_copy(ed.at[pl.ds(base + b * NB, NB)],
                                      idx_d[p], isem[p]).wait()

            def process(p):
                for k in range(NB):
                    pltpu.async_copy(ones_v, acc.at[idx_d[p].at[k]],
                                     ssem, add=True)
                for k in range(NB):
                    pltpu.make_async_copy(ones_v, acc.at[idx_d[p].at[k]],
                                          ssem).wait()

            load_block(0, 0)

            def sup(t, carry):
                b0 = 2 * t
                wait_block(b0, 0)
                load_block(b0 + 1, 1)
                process(0)
                wait_block(b0 + 1, 1)

                @pl.when(t + 1 < nsuper)
                def _():
                    load_block(b0 + 2, 0)
                process(1)
                return carry

            lax.fori_loop(0, nsuper, sup, 0)

        @pl.when(cid == 0)
        def _():
            run_dir(e_d0)

        @pl.when(cid == 1)
        def _():
            run_dir(e_d1)

        plsc.subcore_barrier()

        @pl.when(cid == 0)
        def _():
            pltpu.sync_copy(acc.at[rslice], out0.at[rslice])

        @pl.when(cid == 1)
        def _():
            pltpu.sync_copy(acc.at[rslice], out1.at[rslice])

    run = pl.kernel(
        body,
        out_type=[
            jax.ShapeDtypeStruct((NPAD, HID), jnp.float32),
            jax.ShapeDtypeStruct((NPAD, HID), jnp.float32),
        ],
        mesh=_mesh(),
        scratch_types=scratch,
    )
    return run(d0, d1, ones, zz)


BM = 1000


def _row(d):
    return pl.BlockSpec((BM, d), lambda i: (i, 0))


def _full(a):
    return pl.BlockSpec(a.shape, lambda i: (0,) * a.ndim)


def _dense_mid(sum_i, cnt_i, h_i, wl_ui, wr_ui, b_ui,
               sum_u, cnt_u, h_u, wl_iu, wr_iu, b_iu):
    Du = sum_i.shape[1]
    Di = sum_u.shape[1]

    def body(si, ci, hi, wlui, wrui, bui, su, cu, hu, wliu, wriu, biu, oi, ou):
        mi = si[...] * (1.0 / jnp.maximum(ci[...][:, :1], 1.0))
        vi = (jnp.dot(mi, wlui[...], preferred_element_type=jnp.float32)
              + jnp.dot(hi[...], wrui[...], preferred_element_type=jnp.float32)
              + bui[...])
        oi[...] = jnp.maximum(vi, 0.0)
        mu = su[...] * (1.0 / jnp.maximum(cu[...][:, :1], 1.0))
        vu = (jnp.dot(mu, wliu[...], preferred_element_type=jnp.float32)
              + jnp.dot(hu[...], wriu[...], preferred_element_type=jnp.float32)
              + biu[...])
        ou[...] = jnp.maximum(vu, 0.0)

    args = (sum_i, cnt_i, h_i, wl_ui, wr_ui, b_ui,
            sum_u, cnt_u, h_u, wl_iu, wr_iu, b_iu)
    specs = [_row(Du), _row(CW), _row(Di), _full(wl_ui), _full(wr_ui), _full(b_ui),
             _row(Di), _row(CW), _row(Du), _full(wl_iu), _full(wr_iu), _full(b_iu)]
    return pl.pallas_call(
        body,
        grid=(N // BM,),
        in_specs=specs,
        out_specs=[_row(HID), _row(HID)],
        out_shape=[jax.ShapeDtypeStruct((N, HID), jnp.float32)] * 2,
    )(*args)


def _dense_final(sum_i, cnt_i, h_i, wl_ui, wr_ui, b_ui,
                 sum_u, cnt_u, h_u, wl_iu, wr_iu, b_iu,
                 wh_u, bh_u, wh_i, bh_i):
    Du = sum_i.shape[1]
    Di = sum_u.shape[1]

    def body(si, ci, hi, wlui, wrui, bui, su, cu, hu, wliu, wriu, biu,
             whu, bhu, whi, bhi, zu, zi):
        mi = si[...] * (1.0 / jnp.maximum(ci[...][:, :1], 1.0))
        vi = (jnp.dot(mi, wlui[...], preferred_element_type=jnp.float32)
              + jnp.dot(hi[...], wrui[...], preferred_element_type=jnp.float32)
              + bui[...])
        hi_new = jnp.maximum(vi, 0.0)
        zi[...] = jnp.dot(hi_new, whi[...], preferred_element_type=jnp.float32) + bhi[...]
        mu = su[...] * (1.0 / jnp.maximum(cu[...][:, :1], 1.0))
        vu = (jnp.dot(mu, wliu[...], preferred_element_type=jnp.float32)
              + jnp.dot(hu[...], wriu[...], preferred_element_type=jnp.float32)
              + biu[...])
        hu_new = jnp.maximum(vu, 0.0)
        zu[...] = jnp.dot(hu_new, whu[...], preferred_element_type=jnp.float32) + bhu[...]

    args = (sum_i, cnt_i, h_i, wl_ui, wr_ui, b_ui,
            sum_u, cnt_u, h_u, wl_iu, wr_iu, b_iu,
            wh_u, bh_u, wh_i, bh_i)
    specs = [_row(Du), _row(CW), _row(Di), _full(wl_ui), _full(wr_ui), _full(b_ui),
             _row(Di), _row(CW), _row(Du), _full(wl_iu), _full(wr_iu), _full(b_iu),
             _full(wh_u), _full(bh_u), _full(wh_i), _full(bh_i)]
    return pl.pallas_call(
        body,
        grid=(N // BM,),
        in_specs=specs,
        out_specs=[_row(HID), _row(HID)],
        out_shape=[jax.ShapeDtypeStruct((N, HID), jnp.float32)] * 2,
    )(*args)


def kernel(x_user, edge_index_ui, edge_index_iu, params):
    s_ui, d_ui = _pad_edges(edge_index_ui)
    s_iu, d_iu = _pad_edges(edge_index_iu)

    cnt_i, cnt_u = _counts_pair(d_ui, d_iu)
    cnt_i = cnt_i[:N, :CW]
    cnt_u = cnt_u[:N, :CW]

    h_u = x_user
    h_i = params["emb_item"]
    layers = params["layers"]

    for li in range(len(layers)):
        p = layers[li]
        Di = h_i.shape[1]
        tab_i = h_i if Di == HID else jnp.pad(h_i, ((0, 0), (0, HID - Di)))
        sum_i, sum_u = _segsum_pair(h_u, s_ui, d_ui, tab_i, s_iu, d_iu)
        sum_i = sum_i[:N]
        sum_u = sum_u[:N, :Di]
        a = (sum_i, cnt_i, h_i, p["ui"]["W_l"].T, p["ui"]["W_r"].T,
             p["ui"]["b_l"][None, :],
             sum_u, cnt_u, h_u, p["iu"]["W_l"].T, p["iu"]["W_r"].T,
             p["iu"]["b_l"][None, :])
        if li + 1 < len(layers):
            h_i, h_u = _dense_mid(*a)
        else:
            z_u, z_i = _dense_final(
                *a,
                params["lin_user"]["W"].T, params["lin_user"]["b"][None, :],
                params["lin_item"]["W"].T, params["lin_item"]["b"][None, :])
    return z_u, z_i
